# baseline (speedup 1.0000x reference)
import functools

import jax
import jax.numpy as jnp
from jax import lax
from jax.experimental import pallas as pl
from jax.experimental.pallas import tpu as pltpu
from jax.experimental.pallas import tpu_sc as plsc

N = 10000
E = 320000
D = 128
DOUT = 40

NPAD = 10112
EPAD = 327680
EROWS = 2560
ROWS_PER_TILE = 80
ACC_ROWS_PER_TILE = NPAD // 16
NBLK = 80

DEG_ROWS = 80

_f32 = jnp.float32
_i32 = jnp.int32


def _mesh():
    return plsc.VectorSubcoreMesh(core_axis_name="c", subcore_axis_name="s")


def _make_deg():
    @functools.partial(
        pl.kernel,
        mesh=_mesh(),
        compiler_params=pltpu.CompilerParams(needs_layout_passes=False),
        out_type=jax.ShapeDtypeStruct((2, DEG_ROWS, 128), _f32),
        scratch_types=[
            pltpu.VMEM((ROWS_PER_TILE, 128), _i32),
            pltpu.VMEM((DEG_ROWS, 128), _f32),
            pltpu.VMEM((DEG_ROWS,), _i32),
            pltpu.VMEM_SHARED((DEG_ROWS, 128), _f32),
        ],
    )
    def deg_kernel(dst_hbm, out_hbm, dstv, hist, rowidx, sdeg):
        cid = lax.axis_index("c")
        sid = lax.axis_index("s")

        zeros16 = jnp.zeros((16,), _f32)

        def _zero(i, c):
            for k in range(8):
                hist[i, pl.ds(k * 16, 16)] = zeros16
            return c
        lax.fori_loop(0, DEG_ROWS, _zero, 0)

        for k in range(DEG_ROWS // 16):
            rowidx[pl.ds(k * 16, 16)] = k * 16 + lax.iota(_i32, 16)

        @pl.when(sid == 0)
        def _():
            pltpu.sync_copy(hist, sdeg)

        base = cid * (EROWS // 2) + sid * ROWS_PER_TILE
        pltpu.sync_copy(dst_hbm.at[pl.ds(base, ROWS_PER_TILE)], dstv)
        plsc.subcore_barrier()

        ones = jnp.ones((16,), _f32)

        def _hist(i, c):
            for k in range(8):
                idx = dstv[i, pl.ds(k * 16, 16)]
                hi = lax.shift_right_logical(idx, 7)
                lo = lax.bitwise_and(idx, 127)
                plsc.addupdate_scatter(hist, [hi, lo], ones)
            return c
        lax.fori_loop(0, ROWS_PER_TILE, _hist, 0)

        plsc.subcore_barrier()
        pltpu.sync_copy(hist, sdeg.at[rowidx], add=True)
        plsc.subcore_barrier()

        @pl.when(sid == 0)
        def _():
            pltpu.sync_copy(sdeg, out_hbm.at[cid])

    return deg_kernel


KB = 64
NBLK64 = EPAD // 32 // KB
E64ROWS = EPAD // KB
CH = 8
NPAIRS = NBLK64 // (2 * CH)


def _make_scatter(width):
    @functools.partial(
        pl.kernel,
        mesh=_mesh(),
        compiler_params=pltpu.CompilerParams(needs_layout_passes=False),
        out_type=jax.ShapeDtypeStruct((2, NPAD, width), _f32),
        scratch_types=[
            pltpu.VMEM((CH, KB), _i32),
            pltpu.VMEM((CH, KB), _i32),
            pltpu.VMEM((CH, KB), _i32),
            pltpu.VMEM((CH, KB), _i32),
            pltpu.VMEM((KB, width), _f32),
            pltpu.VMEM((KB, width), _f32),
            pltpu.VMEM((KB, width), _f32),
            pltpu.VMEM((KB, width), _f32),
            pltpu.SemaphoreType.DMA,
            pltpu.SemaphoreType.DMA,
            pltpu.SemaphoreType.DMA,
            pltpu.SemaphoreType.DMA,
            pltpu.SemaphoreType.DMA,
            pltpu.SemaphoreType.DMA,
            pltpu.SemaphoreType.DMA,
            pltpu.SemaphoreType.DMA,
            pltpu.SemaphoreType.DMA,
            pltpu.SemaphoreType.DMA,
            pltpu.VMEM_SHARED((NPAD, width), _f32),
        ],
    )
    def scat_kernel(ht_hbm, src_hbm, dst_hbm, out_hbm,
                    sc0, sc1, dc0, dc1, r0, r1, r2, r3,
                    g0, g1, g2, g3, s0, s1, s2, s3, i0, i1, acc):
        rows = (r0, r1, r2, r3)
        gsems = (g0, g1, g2, g3)
        ssems = (s0, s1, s2, s3)
        srcs = (sc0, sc1)
        dsts = (dc0, dc1)
        isems = (i0, i1)
        cid = lax.axis_index("c")
        sid = lax.axis_index("s")

        pltpu.sync_copy(ht_hbm.at[pl.ds(sid * ACC_ROWS_PER_TILE,
                                        ACC_ROWS_PER_TILE)],
                        acc.at[pl.ds(sid * ACC_ROWS_PER_TILE,
                                     ACC_ROWS_PER_TILE)])

        base = cid * (E64ROWS // 2) + sid * NBLK64

        def i_start(p, m):
            pltpu.async_copy(src_hbm.at[pl.ds(base + CH * m, CH)],
                             srcs[p], isems[p])
            pltpu.async_copy(dst_hbm.at[pl.ds(base + CH * m, CH)],
                             dsts[p], isems[p])

        def i_wait(p):
            pltpu.make_async_copy(src_hbm.at[pl.ds(base, CH)], srcs[p],
                                  isems[p]).wait()
            pltpu.make_async_copy(dst_hbm.at[pl.ds(base, CH)], dsts[p],
                                  isems[p]).wait()

        def g_start(slot, p, r):
            pltpu.async_copy(ht_hbm.at[srcs[p].at[r]], rows[slot],
                             gsems[slot])

        def g_wait(slot):
            pltpu.make_async_copy(ht_hbm.at[srcs[0].at[0]], rows[slot],
                                  gsems[slot]).wait()

        def s_start(slot, p, r):
            pltpu.async_copy(rows[slot], acc.at[dsts[p].at[r]],
                             ssems[slot], add=True)

        def s_wait(slot):
            pltpu.make_async_copy(rows[slot], acc.at[dsts[0].at[0]],
                                  ssems[slot]).wait()

        i_start(0, 0)
        i_wait(0)
        i_start(1, 1)
        plsc.subcore_barrier()
        g_start(0, 0, 0)
        g_start(1, 0, 1)

        def _chunk(j, p, first, last):
            for b in range(CH):
                slot = b % 4
                old = (b + 2) % 4
                g_wait(slot)
                s_start(slot, p, b)
                if first:
                    if b < 2:
                        @pl.when(j > 0)
                        def _(old=old):
                            s_wait(old)
                    else:
                        s_wait(old)
                else:
                    s_wait(old)
                if b == 2:
                    if first and last:
                        pass
                    elif first:
                        @pl.when(j > 0)
                        def _():
                            i_start(1, 2 * j + 1)
                    else:
                        @pl.when(j < NPAIRS - 1)
                        def _():
                            i_start(0, 2 * j + 2)
                if b < CH - 2:
                    g_start(old, p, b + 2)
                elif not last:
                    if b == CH - 2:
                        i_wait(1 - p)
                        g_start(old, 1 - p, 0)
                    else:
                        g_start(old, 1 - p, 1)
                else:
                    @pl.when(j < NPAIRS - 1)
                    def _(old=old, b=b):
                        if b == CH - 2:
                            i_wait(1 - p)
                            g_start(old, 1 - p, 0)
                        else:
                            g_start(old, 1 - p, 1)
            return None

        def _pair(j, c):
            _chunk(j, 0, True, False)
            _chunk(j, 1, False, True)
            return c
        lax.fori_loop(0, NPAIRS, _pair, 0)
        s_wait(2)
        s_wait(3)

        plsc.subcore_barrier()
        pltpu.sync_copy(acc.at[pl.ds(sid * ACC_ROWS_PER_TILE,
                                     ACC_ROWS_PER_TILE)],
                        out_hbm.at[cid, pl.ds(sid * ACC_ROWS_PER_TILE,
                                              ACC_ROWS_PER_TILE)])

    return scat_kernel


def _dinv(degc):
    return lax.rsqrt(degc[0] + degc[1] + 1.0)


def _tc0_body(x_ref, w_ref, degc_ref, out_ref):
    dinv = _dinv(degc_ref[...])
    h = jnp.dot(x_ref[...], w_ref[...], preferred_element_type=_f32)
    out_ref[:N, :] = h * dinv
    out_ref[N:, :] = jnp.zeros((NPAD - N, D), _f32)


def _tc0(x, W1, degc):
    return pl.pallas_call(
        _tc0_body,
        out_shape=jax.ShapeDtypeStruct((NPAD, D), _f32),
    )(x, W1, degc)


def _tcmid_body(wout, parts_ref, ht_ref, degc_ref, w_ref, b_ref, g_ref,
                be_ref, out_ref):
    dinv = _dinv(degc_ref[...])
    s = (parts_ref[0] + parts_ref[1] - ht_ref[...])[:N, :]
    o = s * dinv + b_ref[...]
    mean = jnp.mean(o, axis=0, keepdims=True)
    var = jnp.mean((o - mean) ** 2, axis=0, keepdims=True)
    xn = (o - mean) * lax.rsqrt(var + 1e-5) * g_ref[...] + be_ref[...]
    r = jnp.maximum(xn, 0.0)
    h = jnp.dot(r, w_ref[...], preferred_element_type=_f32)
    out_ref[:N, :] = h * dinv
    out_ref[N:, :] = jnp.zeros((NPAD - N, wout), _f32)


def _tcmid(parts, ht, degc, W, b, g, be, wout):
    return pl.pallas_call(
        functools.partial(_tcmid_body, wout),
        out_shape=jax.ShapeDtypeStruct((NPAD, wout), _f32),
    )(parts, ht, degc, W, b, g, be)


def _tcfin_body(parts_ref, ht_ref, degc_ref, b_ref, out_ref):
    dinv = _dinv(degc_ref[...])
    s = (parts_ref[0, :, :DOUT] + parts_ref[1, :, :DOUT]
         - ht_ref[:, :DOUT])[:N, :]
    o = s * dinv + b_ref[...]
    m = jnp.max(o, axis=1, keepdims=True)
    y = o - m
    out_ref[...] = y - jnp.log(jnp.sum(jnp.exp(y), axis=1, keepdims=True))


def _tcfin(parts, ht, degc, b3):
    return pl.pallas_call(
        _tcfin_body,
        out_shape=jax.ShapeDtypeStruct((N, DOUT), _f32),
    )(parts, ht, degc, b3)


def kernel(x, edge_index, W1, b1, g1, be1, W2, b2, g2, be2, W3, b3):
    src = edge_index[0].astype(_i32)
    dst = edge_index[1].astype(_i32)
    pad_ids = N + (jnp.arange(EPAD - E, dtype=_i32) % 112)
    srcp = jnp.concatenate([src, pad_ids]).reshape(E64ROWS, KB)
    dstp128 = jnp.concatenate([dst, pad_ids]).reshape(EROWS, 128)
    dstp = dstp128.reshape(E64ROWS, KB)
    degp = _make_deg()(dstp128)
    degc = degp.reshape(2, DEG_ROWS * 128)[:, :N].reshape(2, N, 1)

    b1r, g1r, be1r = b1.reshape(1, D), g1.reshape(1, D), be1.reshape(1, D)
    b2r, g2r, be2r = b2.reshape(1, D), g2.reshape(1, D), be2.reshape(1, D)
    W3p = jnp.pad(W3, ((0, 0), (0, D - DOUT)))
    b3r = b3.reshape(1, DOUT)

    scat = _make_scatter(D)

    ht1 = _tc0(x, W1, degc)
    p1 = scat(ht1, srcp, dstp)
    ht2 = _tcmid(p1, ht1, degc, W2, b1r, g1r, be1r, D)
    p2 = scat(ht2, srcp, dstp)
    ht3 = _tcmid(p2, ht2, degc, W3p, b2r, g2r, be2r, D)
    p3 = scat(ht3, srcp, dstp)
    return _tcfin(p3, ht3, degc, b3r)


# speedup vs baseline: 1.1063x; 1.1063x over previous
import functools

import jax
import jax.numpy as jnp
from jax import lax
from jax.experimental import pallas as pl
from jax.experimental.pallas import tpu as pltpu
from jax.experimental.pallas import tpu_sc as plsc

N = 10000
E = 320000
D = 128
DOUT = 40

NPAD = 10112
EPAD = 327680
EROWS = 2560
ROWS_PER_TILE = 80
ACC_ROWS_PER_TILE = NPAD // 16
NBLK = 80

DEG_ROWS = 80

_f32 = jnp.float32
_i32 = jnp.int32


def _mesh():
    return plsc.VectorSubcoreMesh(core_axis_name="c", subcore_axis_name="s")


def _make_deg():
    @functools.partial(
        pl.kernel,
        mesh=_mesh(),
        compiler_params=pltpu.CompilerParams(needs_layout_passes=False),
        out_type=jax.ShapeDtypeStruct((2, DEG_ROWS, 128), _f32),
        scratch_types=[
            pltpu.VMEM((ROWS_PER_TILE, 128), _i32),
            pltpu.VMEM((DEG_ROWS, 128), _f32),
            pltpu.VMEM((DEG_ROWS,), _i32),
            pltpu.VMEM_SHARED((DEG_ROWS, 128), _f32),
        ],
    )
    def deg_kernel(dst_hbm, out_hbm, dstv, hist, rowidx, sdeg):
        cid = lax.axis_index("c")
        sid = lax.axis_index("s")

        zeros16 = jnp.zeros((16,), _f32)

        def _zero(i, c):
            for k in range(8):
                hist[i, pl.ds(k * 16, 16)] = zeros16
            return c
        lax.fori_loop(0, DEG_ROWS, _zero, 0)

        for k in range(DEG_ROWS // 16):
            rowidx[pl.ds(k * 16, 16)] = k * 16 + lax.iota(_i32, 16)

        @pl.when(sid == 0)
        def _():
            pltpu.sync_copy(hist, sdeg)

        base = cid * (EROWS // 2) + sid * ROWS_PER_TILE
        pltpu.sync_copy(dst_hbm.at[pl.ds(base, ROWS_PER_TILE)], dstv)
        plsc.subcore_barrier()

        ones = jnp.ones((16,), _f32)

        def _hist(i, c):
            for k in range(8):
                idx = dstv[i, pl.ds(k * 16, 16)]
                hi = lax.shift_right_logical(idx, 7)
                lo = lax.bitwise_and(idx, 127)
                plsc.addupdate_scatter(hist, [hi, lo], ones)
            return c
        lax.fori_loop(0, ROWS_PER_TILE, _hist, 0)

        plsc.subcore_barrier()
        pltpu.sync_copy(hist, sdeg.at[rowidx], add=True)
        plsc.subcore_barrier()

        @pl.when(sid == 0)
        def _():
            pltpu.sync_copy(sdeg, out_hbm.at[cid])

    return deg_kernel


CH = 8
NPAIR = NBLK // (2 * CH)


def _make_scatter(width):
    @functools.partial(
        pl.kernel,
        mesh=_mesh(),
        compiler_params=pltpu.CompilerParams(needs_layout_passes=False),
        out_type=jax.ShapeDtypeStruct((2, NPAD, width), _f32),
        scratch_types=[
            pltpu.VMEM((NBLK, 128), _i32),
            pltpu.VMEM((CH, 128), _i32),
            pltpu.VMEM((CH, 128), _i32),
            pltpu.VMEM((128, width), _f32),
            pltpu.VMEM((128, width), _f32),
            pltpu.SemaphoreType.DMA,
            pltpu.SemaphoreType.DMA,
            pltpu.SemaphoreType.DMA,
            pltpu.SemaphoreType.DMA,
            pltpu.SemaphoreType.DMA,
            pltpu.SemaphoreType.DMA,
            pltpu.VMEM_SHARED((NPAD, width), _f32),
        ],
    )
    def scat_kernel(ht_hbm, src_hbm, dst_hbm, out_hbm,
                    dst_idx, src_c0, src_c1, rows0, rows1,
                    gsem0, gsem1, ssem0, ssem1, isem0, isem1, acc):
        rows = (rows0, rows1)
        gsems = (gsem0, gsem1)
        ssems = (ssem0, ssem1)
        srcs = (src_c0, src_c1)
        isems = (isem0, isem1)
        cid = lax.axis_index("c")
        sid = lax.axis_index("s")

        pltpu.async_copy(ht_hbm.at[pl.ds(sid * ACC_ROWS_PER_TILE,
                                         ACC_ROWS_PER_TILE)],
                         acc.at[pl.ds(sid * ACC_ROWS_PER_TILE,
                                      ACC_ROWS_PER_TILE)], gsem0)

        base = cid * (EROWS // 2) + sid * NBLK

        def i_start(p, m):
            pltpu.async_copy(src_hbm.at[pl.ds(base + CH * m, CH)],
                             srcs[p], isems[p])

        def i_wait(p):
            pltpu.make_async_copy(src_hbm.at[pl.ds(base, CH)], srcs[p],
                                  isems[p]).wait()

        def g_start(slot, p, r):
            pltpu.async_copy(ht_hbm.at[srcs[p].at[r]], rows[slot],
                             gsems[slot])

        def g_wait(slot):
            pltpu.make_async_copy(ht_hbm.at[srcs[0].at[0]], rows[slot],
                                  gsems[slot]).wait()

        def s_start(slot, b):
            pltpu.async_copy(rows[slot], acc.at[dst_idx.at[b]],
                             ssems[slot], add=True)

        def s_wait(slot):
            pltpu.make_async_copy(rows[slot], acc.at[dst_idx.at[0]],
                                  ssems[slot]).wait()

        pltpu.sync_copy(dst_hbm.at[pl.ds(base, NBLK)], dst_idx)
        i_start(0, 0)
        i_wait(0)
        i_start(1, 1)
        pltpu.make_async_copy(
            ht_hbm.at[pl.ds(sid * ACC_ROWS_PER_TILE, ACC_ROWS_PER_TILE)],
            acc.at[pl.ds(sid * ACC_ROWS_PER_TILE, ACC_ROWS_PER_TILE)],
            gsem0).wait()
        plsc.subcore_barrier()
        g_start(0, 0, 0)
        g_start(1, 0, 1)

        def _pair(j, c):
            @pl.when(j > 0)
            def _():
                i_start(1, 2 * j + 1)
            for b in range(CH):
                slot = b % 2
                g_wait(slot)
                s_start(slot, 16 * j + b)
                s_wait(slot)
                if b == CH - 2:
                    i_wait(1)
                if b < CH - 2:
                    g_start(slot, 0, b + 2)
                else:
                    g_start(slot, 1, b - (CH - 2))
            @pl.when(j < NPAIR - 1)
            def _():
                i_start(0, 2 * j + 2)
            for b in range(CH):
                slot = b % 2
                g_wait(slot)
                s_start(slot, 16 * j + CH + b)
                s_wait(slot)
                if b < CH - 2:
                    g_start(slot, 1, b + 2)
                elif b == CH - 2:
                    @pl.when(j < NPAIR - 1)
                    def _(slot=slot):
                        i_wait(0)
                        g_start(slot, 0, 0)
                else:
                    @pl.when(j < NPAIR - 1)
                    def _(slot=slot):
                        g_start(slot, 0, 1)
            return c
        lax.fori_loop(0, NPAIR, _pair, 0)

        plsc.subcore_barrier()
        pltpu.sync_copy(acc.at[pl.ds(sid * ACC_ROWS_PER_TILE,
                                     ACC_ROWS_PER_TILE)],
                        out_hbm.at[cid, pl.ds(sid * ACC_ROWS_PER_TILE,
                                              ACC_ROWS_PER_TILE)])

    return scat_kernel


def _dinv(degc):
    return lax.rsqrt(degc[0] + degc[1] + 1.0)


def _tc0_body(x_ref, w_ref, degc_ref, out_ref):
    dinv = _dinv(degc_ref[...])
    h = jnp.dot(x_ref[...], w_ref[...], preferred_element_type=_f32)
    out_ref[:N, :] = h * dinv
    out_ref[N:, :] = jnp.zeros((NPAD - N, D), _f32)


def _tc0(x, W1, degc):
    return pl.pallas_call(
        _tc0_body,
        out_shape=jax.ShapeDtypeStruct((NPAD, D), _f32),
    )(x, W1, degc)


def _tcmid_body(wout, parts_ref, ht_ref, degc_ref, w_ref, b_ref, g_ref,
                be_ref, out_ref):
    dinv = _dinv(degc_ref[...])
    s = (parts_ref[0] + parts_ref[1] - ht_ref[...])[:N, :]
    o = s * dinv + b_ref[...]
    mean = jnp.mean(o, axis=0, keepdims=True)
    var = jnp.mean((o - mean) ** 2, axis=0, keepdims=True)
    xn = (o - mean) * lax.rsqrt(var + 1e-5) * g_ref[...] + be_ref[...]
    r = jnp.maximum(xn, 0.0)
    h = jnp.dot(r, w_ref[...], preferred_element_type=_f32)
    out_ref[:N, :] = h * dinv
    out_ref[N:, :] = jnp.zeros((NPAD - N, wout), _f32)


def _tcmid(parts, ht, degc, W, b, g, be, wout):
    return pl.pallas_call(
        functools.partial(_tcmid_body, wout),
        out_shape=jax.ShapeDtypeStruct((NPAD, wout), _f32),
    )(parts, ht, degc, W, b, g, be)


def _tcfin_body(parts_ref, ht_ref, degc_ref, b_ref, out_ref):
    dinv = _dinv(degc_ref[...])
    s = (parts_ref[0, :, :DOUT] + parts_ref[1, :, :DOUT]
         - ht_ref[:, :DOUT])[:N, :]
    o = s * dinv + b_ref[...]
    m = jnp.max(o, axis=1, keepdims=True)
    y = o - m
    out_ref[...] = y - jnp.log(jnp.sum(jnp.exp(y), axis=1, keepdims=True))


def _tcfin(parts, ht, degc, b3):
    return pl.pallas_call(
        _tcfin_body,
        out_shape=jax.ShapeDtypeStruct((N, DOUT), _f32),
    )(parts, ht, degc, b3)


def kernel(x, edge_index, W1, b1, g1, be1, W2, b2, g2, be2, W3, b3):
    src = edge_index[0].astype(_i32)
    dst = edge_index[1].astype(_i32)
    pad_ids = N + (jnp.arange(EPAD - E, dtype=_i32) % 112)
    srcp = jnp.concatenate([src, pad_ids]).reshape(EROWS, 128)
    dstp = jnp.concatenate([dst, pad_ids]).reshape(EROWS, 128)
    degp = _make_deg()(dstp)
    degc = degp.reshape(2, DEG_ROWS * 128)[:, :N].reshape(2, N, 1)

    b1r, g1r, be1r = b1.reshape(1, D), g1.reshape(1, D), be1.reshape(1, D)
    b2r, g2r, be2r = b2.reshape(1, D), g2.reshape(1, D), be2.reshape(1, D)
    W3p = jnp.pad(W3, ((0, 0), (0, D - DOUT)))
    b3r = b3.reshape(1, DOUT)

    scat = _make_scatter(D)

    ht1 = _tc0(x, W1, degc)
    p1 = scat(ht1, srcp, dstp)
    ht2 = _tcmid(p1, ht1, degc, W2, b1r, g1r, be1r, D)
    p2 = scat(ht2, srcp, dstp)
    ht3 = _tcmid(p2, ht2, degc, W3p, b2r, g2r, be2r, D)
    p3 = scat(ht3, srcp, dstp)
    return _tcfin(p3, ht3, degc, b3r)
